# baseline (speedup 1.0000x reference)
import functools

import jax
import jax.numpy as jnp
from jax import lax
from jax.experimental import pallas as pl
from jax.experimental.pallas import tpu as pltpu
from jax.experimental.pallas import tpu_sc as plsc

D = 64
ROWS = 50
NBUF = 8


def _make_sc_gather(n_batches, NW):
    per_w = n_batches // NW
    n_groups = per_w // NBUF
    assert per_w % NBUF == 0
    n_idx = per_w * ROWS
    full, rem = divmod(ROWS, 16)

    mesh = plsc.VectorSubcoreMesh(core_axis_name="c", subcore_axis_name="s")
    info = plsc.get_sparse_core_info()
    NC = info.num_cores

    scratch = (
        [pltpu.VMEM((n_idx + 16,), jnp.int32)]
        + [pltpu.VMEM((ROWS, D), jnp.float32) for _ in range(NBUF)]
        + [pltpu.SemaphoreType.DMA for _ in range(2 * NBUF)]
    )

    @functools.partial(
        pl.kernel,
        mesh=mesh,
        out_type=jax.ShapeDtypeStruct((n_batches, ROWS, D), jnp.float32),
        scratch_types=scratch,
    )
    def k(table3_hbm, idx_hbm, out_hbm, idx_v, *bufs_and_sems):
        bufs = bufs_and_sems[:NBUF]
        gsem = bufs_and_sems[NBUF:2 * NBUF]
        wsem = bufs_and_sems[2 * NBUF:]

        wid = lax.axis_index("s") * NC + lax.axis_index("c")
        batch0 = wid * per_w

        pltpu.sync_copy(idx_hbm.at[wid], idx_v.at[pl.ds(0, n_idx)])

        def issue_batch(b, slot):
            base = b * ROWS
            for g in range(full + 1):
                iv = idx_v[pl.ds(base + g * 16, 16)]
                qv = jax.lax.shift_right_logical(iv, 3)
                kv = jax.lax.bitwise_and(iv, 7)
                lanes = 16 if g < full else rem
                for lane in range(lanes):
                    s = g * 16 + lane
                    pltpu.async_copy(
                        table3_hbm.at[qv[lane], pl.ds(kv[lane], 1)],
                        bufs[slot].at[pl.ds(s, 1)],
                        gsem[slot],
                    )

        def drain_batch(slot):
            pltpu.make_async_copy(out_hbm.at[0], bufs[slot], gsem[slot]).wait()

        def write_batch(b, slot, make):
            f = pltpu.make_async_copy if make else pltpu.async_copy
            return f(bufs[slot], out_hbm.at[batch0 + b], wsem[slot])

        for slot in range(NBUF):
            issue_batch(slot, slot)

        @pl.loop(0, n_groups - 1)
        def _(g):
            b0 = g * NBUF
            for slot in range(NBUF):
                drain_batch(slot)
                write_batch(b0 + slot, slot, make=False)
            for slot in range(NBUF):
                write_batch(b0 + slot, slot, make=True).wait()
                issue_batch(b0 + NBUF + slot, slot)

        b0 = (n_groups - 1) * NBUF
        for slot in range(NBUF):
            drain_batch(slot)
            write_batch(b0 + slot, slot, make=False)
        for slot in range(NBUF):
            write_batch(b0 + slot, slot, make=True).wait()

    return k


def kernel(inputs, embedding):
    R, C = inputs.shape
    info = plsc.get_sparse_core_info()
    NW = info.num_cores * info.num_subcores
    idx = inputs.reshape(NW, (R // NW) * C).astype(jnp.int32)
    V = embedding.shape[0]
    table3 = embedding.reshape(V // 8, 8, D)
    return _make_sc_gather(R, NW)(table3, idx)


# speedup vs baseline: 1.0226x; 1.0226x over previous
import functools

import jax
import jax.numpy as jnp
from jax import lax
from jax.experimental import pallas as pl
from jax.experimental.pallas import tpu as pltpu
from jax.experimental.pallas import tpu_sc as plsc

D = 64
ROWS = 50
NBUF = 4


def _make_sc_gather(n_batches, NW):
    per_w = n_batches // NW
    n_groups = per_w // NBUF
    assert per_w % NBUF == 0
    n_idx = per_w * ROWS
    full, rem = divmod(ROWS, 16)

    mesh = plsc.VectorSubcoreMesh(core_axis_name="c", subcore_axis_name="s")
    info = plsc.get_sparse_core_info()
    NC = info.num_cores

    scratch = (
        [pltpu.VMEM((n_idx + 16,), jnp.int32)]
        + [pltpu.VMEM((ROWS, D), jnp.float32) for _ in range(NBUF)]
        + [pltpu.SemaphoreType.DMA for _ in range(2 * NBUF)]
    )

    @functools.partial(
        pl.kernel,
        mesh=mesh,
        out_type=jax.ShapeDtypeStruct((n_batches, ROWS, D), jnp.float32),
        scratch_types=scratch,
    )
    def k(table3_hbm, idx_hbm, out_hbm, idx_v, *bufs_and_sems):
        bufs = bufs_and_sems[:NBUF]
        gsem = bufs_and_sems[NBUF:2 * NBUF]
        wsem = bufs_and_sems[2 * NBUF:]

        wid = lax.axis_index("s") * NC + lax.axis_index("c")
        batch0 = wid * per_w

        pltpu.sync_copy(idx_hbm.at[wid], idx_v.at[pl.ds(0, n_idx)])

        def issue_batch(b, slot):
            base = b * ROWS
            for g in range(full + 1):
                iv = idx_v[pl.ds(base + g * 16, 16)]
                qv = jax.lax.shift_right_logical(iv, 3)
                kv = jax.lax.bitwise_and(iv, 7)
                lanes = 16 if g < full else rem
                for lane in range(lanes):
                    s = g * 16 + lane
                    pltpu.async_copy(
                        table3_hbm.at[qv[lane], pl.ds(kv[lane], 1)],
                        bufs[slot].at[pl.ds(s, 1)],
                        gsem[slot],
                    )

        def drain_batch(slot):
            pltpu.make_async_copy(out_hbm.at[0], bufs[slot], gsem[slot]).wait()

        def write_batch(b, slot, make):
            f = pltpu.make_async_copy if make else pltpu.async_copy
            return f(bufs[slot], out_hbm.at[batch0 + b], wsem[slot])

        for slot in range(NBUF):
            issue_batch(slot, slot)

        @pl.loop(0, n_groups - 1)
        def _(g):
            b0 = g * NBUF
            for slot in range(NBUF):
                drain_batch(slot)
                write_batch(b0 + slot, slot, make=False)
            for slot in range(NBUF):
                write_batch(b0 + slot, slot, make=True).wait()
                issue_batch(b0 + NBUF + slot, slot)

        b0 = (n_groups - 1) * NBUF
        for slot in range(NBUF):
            drain_batch(slot)
            write_batch(b0 + slot, slot, make=False)
        for slot in range(NBUF):
            write_batch(b0 + slot, slot, make=True).wait()

    return k


def kernel(inputs, embedding):
    R, C = inputs.shape
    info = plsc.get_sparse_core_info()
    NW = info.num_cores * info.num_subcores
    idx = inputs.reshape(NW, (R // NW) * C).astype(jnp.int32)
    V = embedding.shape[0]
    table3 = embedding.reshape(V // 8, 8, D)
    return _make_sc_gather(R, NW)(table3, idx)
